# Initial kernel scaffold; baseline (speedup 1.0000x reference)
import jax
import jax.numpy as jnp
from jax.experimental import pallas as pl


def kernel(tam_indices, emb_table, W, b):
    raise NotImplementedError("write your pallas kernel here")


# speedup vs baseline: 7.1711x; 7.1711x over previous
import functools

import numpy as np
import jax
import jax.numpy as jnp
from jax import lax
from jax.experimental import pallas as pl
from jax.experimental.pallas import tpu as pltpu
from jax.experimental.pallas import tpu_sc as plsc

VOCAB = 1000000
E = 32
F = 26
B = 4096
S = 50
ELEM_PER_B = S * F
CHUNK_B = 2
CHUNK_TOK = CHUNK_B * S
CHUNK_IDX = CHUNK_B * ELEM_PER_B
IDX_ROWS = 21
IDX_PAD = IDX_ROWS * 128
NC = 2
NS = 16
NW = NC * NS
NCHUNK = B // CHUNK_B
CHUNK_PER_W = NCHUNK // NW

PROJ_BLK = 8000


def _proj_body(t_ref, w_ref, b_ref, o_ref):
    o_ref[...] = (
        jnp.dot(t_ref[...], w_ref[...], preferred_element_type=jnp.float32)
        + b_ref[...]
    )


def _project(table, w_pad, b_pad):
    return pl.pallas_call(
        _proj_body,
        out_shape=jax.ShapeDtypeStruct((VOCAB, E), jnp.float32),
        grid=(VOCAB // PROJ_BLK,),
        in_specs=[
            pl.BlockSpec((PROJ_BLK, E), lambda i: (i, 0)),
            pl.BlockSpec((E, E), lambda i: (0, 0)),
            pl.BlockSpec((1, E), lambda i: (0, 0)),
        ],
        out_specs=pl.BlockSpec((PROJ_BLK, E), lambda i: (i, 0)),
    )(table, w_pad, b_pad)


def _pos_const():
    t = np.arange(CHUNK_TOK)
    beta, s = t // S, t % S
    f = np.arange(E)
    p = s[:, None] * F + f[None, :]
    pos = (p % S) * F + p // S + beta[:, None] * ELEM_PER_B
    pos = np.where(f[None, :] < F, pos, 0)
    return jnp.asarray(pos.astype(np.int32))


def _sc_body(p_hbm, idx_hbm, pos_hbm, out_hbm, idx_v, rows_v, pos_v, out_v, sem):
    wid = lax.axis_index("s") * NC + lax.axis_index("c")
    base_chunk = wid * CHUNK_PER_W
    mask_hi = lax.iota(jnp.int32, 16) < (F - 16)

    pltpu.sync_copy(pos_hbm, pos_v)

    @pl.loop(0, CHUNK_PER_W)
    def _chunk(ci):
        c = base_chunk + ci
        pltpu.sync_copy(idx_hbm.at[c], idx_v)
        descs = []
        for j in range(IDX_ROWS):
            descs.append(
                pltpu.async_copy(
                    p_hbm.at[idx_v.at[j]],
                    rows_v.at[pl.ds(j * 128, 128)],
                    sem,
                )
            )
        for d in descs:
            d.wait()

        @pl.loop(0, CHUNK_TOK)
        def _tok(t):
            rbase = t * F
            acc_lo = rows_v[rbase, 0:16]
            acc_hi = rows_v[rbase, 16:32]
            for r in range(1, F):
                acc_lo = acc_lo + rows_v[rbase + r, 0:16]
                acc_hi = acc_hi + rows_v[rbase + r, 16:32]
            acc_lo = jnp.maximum(acc_lo, 0.0)
            acc_hi = jnp.maximum(acc_hi, 0.0)
            plsc.store_scatter(out_v, [pos_v[t, 0:16]], acc_lo)
            plsc.store_scatter(out_v, [pos_v[t, 16:32]], acc_hi, mask=mask_hi)

        pltpu.sync_copy(out_v, out_hbm.at[pl.ds(c * CHUNK_IDX, CHUNK_IDX)])


_sc_gather = functools.partial(
    pl.kernel,
    out_type=jax.ShapeDtypeStruct((B * ELEM_PER_B,), jnp.float32),
    mesh=plsc.VectorSubcoreMesh(core_axis_name="c", subcore_axis_name="s"),
    compiler_params=pltpu.CompilerParams(
        needs_layout_passes=False, use_tc_tiling_on_sc=False
    ),
    scratch_types=[
        pltpu.VMEM((IDX_ROWS, 128), jnp.int32),
        pltpu.VMEM((IDX_PAD, E), jnp.float32),
        pltpu.VMEM((CHUNK_TOK, E), jnp.int32),
        pltpu.VMEM((CHUNK_IDX,), jnp.float32),
        pltpu.SemaphoreType.DMA,
    ],
)(_sc_body)


def kernel(tam_indices, emb_table, W, b):
    idx = tam_indices.astype(jnp.int32).reshape(NCHUNK, CHUNK_IDX)
    idx = jnp.pad(idx, ((0, 0), (0, IDX_PAD - CHUNK_IDX)))
    idx = idx.reshape(NCHUNK, IDX_ROWS, 128)
    w_pad = jnp.zeros((E, E), jnp.float32).at[:, :F].set(W.T / F)
    b_pad = jnp.zeros((1, E), jnp.float32).at[0, :F].set(b / F)
    P = _project(emb_table, w_pad, b_pad)
    out = _sc_gather(P, idx, _pos_const())
    return out.reshape(B, S, F)


# speedup vs baseline: 9.5916x; 1.3375x over previous
import functools

import numpy as np
import jax
import jax.numpy as jnp
from jax import lax
from jax.experimental import pallas as pl
from jax.experimental.pallas import tpu as pltpu
from jax.experimental.pallas import tpu_sc as plsc

VOCAB = 1000000
E = 32
F = 26
B = 4096
S = 50
ELEM_PER_B = S * F
CHUNK_B = 2
CHUNK_TOK = CHUNK_B * S
CHUNK_IDX = CHUNK_B * ELEM_PER_B
IDX_ROWS = 21
IDX_PAD = IDX_ROWS * 128
NC = 2
NS = 16
NW = NC * NS
NCHUNK = B // CHUNK_B
CHUNK_PER_W = NCHUNK // NW

PROJ_BLK = 8000


def _proj_body(t_ref, w_ref, b_ref, o_ref):
    o_ref[...] = (
        jnp.dot(
            t_ref[...],
            w_ref[...],
            preferred_element_type=jnp.float32,
            precision=lax.Precision.HIGHEST,
        )
        + b_ref[...]
    ).astype(jnp.bfloat16)


def _project(table, w_pad, b_pad):
    return pl.pallas_call(
        _proj_body,
        out_shape=jax.ShapeDtypeStruct((VOCAB, E), jnp.bfloat16),
        grid=(VOCAB // PROJ_BLK,),
        in_specs=[
            pl.BlockSpec((PROJ_BLK, E), lambda i: (i, 0)),
            pl.BlockSpec((E, E), lambda i: (0, 0)),
            pl.BlockSpec((1, E), lambda i: (0, 0)),
        ],
        out_specs=pl.BlockSpec((PROJ_BLK, E), lambda i: (i, 0)),
    )(table, w_pad, b_pad)


def _pos_const():
    t = np.arange(CHUNK_TOK)
    beta, s = t // S, t % S
    f = np.concatenate([np.arange(0, E, 2), np.arange(1, E, 2)])
    p = s[:, None] * F + f[None, :]
    pos = (p % S) * F + p // S + beta[:, None] * ELEM_PER_B
    pos = np.where(f[None, :] < F, pos, 0)
    return jnp.asarray(pos.astype(np.int32))


def _sc_body(p_hbm, idx_hbm, pos_hbm, out_hbm,
             idx_v, rows_v, pos_v, out_v, sem0, sem1):
    wid = lax.axis_index("s") * NC + lax.axis_index("c")
    base_chunk = wid * CHUNK_PER_W
    mask13 = lax.iota(jnp.int32, 16) < (F // 2)
    sems = (sem0, sem1)

    pltpu.sync_copy(pos_hbm, pos_v)

    def fire(buf, c):
        pltpu.sync_copy(idx_hbm.at[c], idx_v.at[buf])
        for j in range(IDX_ROWS):
            pltpu.async_copy(
                p_hbm.at[idx_v.at[buf].at[j]],
                rows_v.at[buf].at[pl.ds(j * 128, 128)],
                sems[buf],
            )

    def drain(buf):
        for j in range(IDX_ROWS):
            pltpu.make_async_copy(
                p_hbm.at[idx_v.at[buf].at[j]],
                rows_v.at[buf].at[pl.ds(j * 128, 128)],
                sems[buf],
            ).wait()

    def compute(buf, c):
        @pl.loop(0, CHUNK_TOK)
        def _tok(t):
            rbase = t * F
            acc_a, acc_b = plsc.unpack(
                rows_v[buf, rbase, 0:E], format=plsc.PackFormat.INTERLEAVED
            )
            for r in range(1, F):
                xa, xb = plsc.unpack(
                    rows_v[buf, rbase + r, 0:E],
                    format=plsc.PackFormat.INTERLEAVED,
                )
                acc_a = acc_a + xa
                acc_b = acc_b + xb
            acc_a = jnp.maximum(acc_a, 0.0)
            acc_b = jnp.maximum(acc_b, 0.0)
            plsc.store_scatter(out_v, [pos_v[t, 0:16]], acc_a, mask=mask13)
            plsc.store_scatter(out_v, [pos_v[t, 16:32]], acc_b, mask=mask13)

        pltpu.sync_copy(out_v, out_hbm.at[pl.ds(c * CHUNK_IDX, CHUNK_IDX)])

    fire(0, base_chunk)

    @pl.loop(0, CHUNK_PER_W, step=2)
    def _chunk(ci):
        c = base_chunk + ci
        fire(1, c + 1)
        drain(0)
        compute(0, c)

        @pl.when(ci + 2 < CHUNK_PER_W)
        def _():
            fire(0, c + 2)

        drain(1)
        compute(1, c + 1)


_sc_gather = functools.partial(
    pl.kernel,
    out_type=jax.ShapeDtypeStruct((B * ELEM_PER_B,), jnp.float32),
    mesh=plsc.VectorSubcoreMesh(core_axis_name="c", subcore_axis_name="s"),
    compiler_params=pltpu.CompilerParams(
        needs_layout_passes=False, use_tc_tiling_on_sc=False
    ),
    scratch_types=[
        pltpu.VMEM((2, IDX_ROWS, 128), jnp.int32),
        pltpu.VMEM((2, IDX_PAD, E), jnp.bfloat16),
        pltpu.VMEM((CHUNK_TOK, E), jnp.int32),
        pltpu.VMEM((CHUNK_IDX,), jnp.float32),
        pltpu.SemaphoreType.DMA,
        pltpu.SemaphoreType.DMA,
    ],
)(_sc_body)


def kernel(tam_indices, emb_table, W, b):
    idx = tam_indices.astype(jnp.int32).reshape(NCHUNK, CHUNK_IDX)
    idx = jnp.pad(idx, ((0, 0), (0, IDX_PAD - CHUNK_IDX)))
    idx = idx.reshape(NCHUNK, IDX_ROWS, 128)
    w_pad = jnp.zeros((E, E), jnp.float32).at[:, :F].set(W.T / F)
    b_pad = jnp.zeros((1, E), jnp.float32).at[0, :F].set(b / F)
    P = _project(emb_table, w_pad, b_pad)
    out = _sc_gather(P, idx, _pos_const())
    return out.reshape(B, S, F)


# speedup vs baseline: 16.9678x; 1.7690x over previous
import functools

import numpy as np
import jax
import jax.numpy as jnp
from jax import lax
from jax.experimental import pallas as pl
from jax.experimental.pallas import tpu as pltpu
from jax.experimental.pallas import tpu_sc as plsc

VOCAB = 1000000
E = 32
F = 26
B = 4096
S = 50
ELEM_PER_B = S * F
CHUNK_B = 2
CHUNK_TOK = CHUNK_B * S
CHUNK_IDX = CHUNK_B * ELEM_PER_B
IDX_GROUPS = [(j * 128, 128) for j in range(20)] + [(2560, 40)]
NC = 2
NS = 16
NW = NC * NS
NCHUNK = B // CHUNK_B
CHUNK_PER_W = NCHUNK // NW

VOCAB4 = VOCAB // 4
WIDE = 4 * E
PROJ_BLK = 10000


def _proj_body(t_ref, w_ref, b_ref, o_ref):
    o_ref[...] = (
        jnp.dot(
            t_ref[...],
            w_ref[...],
            preferred_element_type=jnp.float32,
            precision=lax.Precision.HIGHEST,
        )
        + b_ref[...]
    ).astype(jnp.bfloat16)


def _project(table_wide, w_blk, b_tile):
    return pl.pallas_call(
        _proj_body,
        out_shape=jax.ShapeDtypeStruct((VOCAB4, WIDE), jnp.bfloat16),
        grid=(VOCAB4 // PROJ_BLK,),
        in_specs=[
            pl.BlockSpec((PROJ_BLK, WIDE), lambda i: (i, 0)),
            pl.BlockSpec((WIDE, WIDE), lambda i: (0, 0)),
            pl.BlockSpec((1, WIDE), lambda i: (0, 0)),
        ],
        out_specs=pl.BlockSpec((PROJ_BLK, WIDE), lambda i: (i, 0)),
    )(table_wide, w_blk, b_tile)


def _pos_const():
    t = np.arange(CHUNK_TOK)
    beta, s = t // S, t % S
    f = np.concatenate([np.arange(0, E, 2), np.arange(1, E, 2)])
    p = s[:, None] * F + f[None, :]
    pos = (p % S) * F + p // S + beta[:, None] * ELEM_PER_B
    pos = np.where(f[None, :] < F, pos, 0)
    return jnp.asarray(pos.astype(np.int32))


def _sc_body(p_hbm, idx_hbm, pos_hbm, out_hbm,
             idx_v, rows_v, pos_v, out_v, sem0, sem1):
    wid = lax.axis_index("s") * NC + lax.axis_index("c")
    base_chunk = wid * CHUNK_PER_W
    mask13 = lax.iota(jnp.int32, 16) < (F // 2)
    sems = (sem0, sem1)

    pltpu.sync_copy(pos_hbm, pos_v)

    def fire(buf, c):
        pltpu.sync_copy(
            idx_hbm.at[pl.ds(c * CHUNK_IDX, CHUNK_IDX)], idx_v.at[buf]
        )
        for off, n in IDX_GROUPS:
            pltpu.async_copy(
                p_hbm.at[idx_v.at[buf].at[pl.ds(off, n)]],
                rows_v.at[buf].at[pl.ds(off, n)],
                sems[buf],
            )

    def drain(buf):
        for off, n in IDX_GROUPS:
            pltpu.make_async_copy(
                p_hbm.at[idx_v.at[buf].at[pl.ds(off, n)]],
                rows_v.at[buf].at[pl.ds(off, n)],
                sems[buf],
            ).wait()

    def compute(buf, c):
        @pl.loop(0, CHUNK_TOK)
        def _tok(t):
            rbase = t * F
            acc_a, acc_b = plsc.unpack(
                rows_v[buf, rbase, 0:E], format=plsc.PackFormat.INTERLEAVED
            )
            for r in range(1, F):
                xa, xb = plsc.unpack(
                    rows_v[buf, rbase + r, 0:E],
                    format=plsc.PackFormat.INTERLEAVED,
                )
                acc_a = acc_a + xa
                acc_b = acc_b + xb
            acc_a = jnp.maximum(acc_a, 0.0)
            acc_b = jnp.maximum(acc_b, 0.0)
            plsc.store_scatter(out_v, [pos_v[t, 0:16]], acc_a, mask=mask13)
            plsc.store_scatter(out_v, [pos_v[t, 16:32]], acc_b, mask=mask13)

        pltpu.sync_copy(out_v, out_hbm.at[pl.ds(c * CHUNK_IDX, CHUNK_IDX)])

    fire(0, base_chunk)

    @pl.loop(0, CHUNK_PER_W, step=2)
    def _chunk(ci):
        c = base_chunk + ci
        fire(1, c + 1)
        drain(0)
        compute(0, c)

        @pl.when(ci + 2 < CHUNK_PER_W)
        def _():
            fire(0, c + 2)

        drain(1)
        compute(1, c + 1)


_sc_gather = functools.partial(
    pl.kernel,
    out_type=jax.ShapeDtypeStruct((B * ELEM_PER_B,), jnp.float32),
    mesh=plsc.VectorSubcoreMesh(core_axis_name="c", subcore_axis_name="s"),
    compiler_params=pltpu.CompilerParams(
        needs_layout_passes=False, use_tc_tiling_on_sc=False
    ),
    scratch_types=[
        pltpu.VMEM((2, CHUNK_IDX), jnp.int32),
        pltpu.VMEM((2, CHUNK_IDX, E), jnp.bfloat16),
        pltpu.VMEM((CHUNK_TOK, E), jnp.int32),
        pltpu.VMEM((CHUNK_IDX,), jnp.float32),
        pltpu.SemaphoreType.DMA,
        pltpu.SemaphoreType.DMA,
    ],
)(_sc_body)


def kernel(tam_indices, emb_table, W, b):
    idx = tam_indices.astype(jnp.int32).reshape(-1)
    w_pad = jnp.zeros((E, E), jnp.float32).at[:, :F].set(W.T / F)
    b_pad = jnp.zeros((1, E), jnp.float32).at[0, :F].set(b / F)
    w_blk = jnp.zeros((WIDE, WIDE), jnp.float32)
    for q in range(4):
        w_blk = w_blk.at[q * E:(q + 1) * E, q * E:(q + 1) * E].set(w_pad)
    b_tile = jnp.tile(b_pad, (1, 4))
    P = _project(emb_table.reshape(VOCAB4, WIDE), w_blk, b_tile)
    P = P.reshape(VOCAB, E)
    out = _sc_gather(P, idx, _pos_const())
    return out.reshape(B, S, F)
